# baseline (speedup 1.0000x reference)
import functools

import jax
import jax.numpy as jnp
from jax import lax
from jax.experimental import pallas as pl
from jax.experimental.pallas import tpu as pltpu
from jax.experimental.pallas import tpu_sc as plsc

NC = 2
NS = 16
NW = NC * NS

CHUNK = 800
PACK_BC = 4096


def _tc_pack_table(table_t):
    d, v = table_t.shape
    bc = PACK_BC
    grid = (v + bc - 1) // bc
    vmain = (grid - 1) * bc
    rem = v - vmain

    def pack_kernel(t_ref, o_hbm, y_ref, sem):
        i = pl.program_id(0)
        y_ref[...] = t_ref[...].T

        @pl.when(i < grid - 1)
        def _full():
            cp = pltpu.make_async_copy(
                y_ref, o_hbm.at[pl.ds(i * bc, bc), :], sem)
            cp.start()
            cp.wait()

        @pl.when(i == grid - 1)
        def _tail():
            cp = pltpu.make_async_copy(
                y_ref.at[pl.ds(0, rem)], o_hbm.at[pl.ds(vmain, rem), :], sem)
            cp.start()
            cp.wait()

    return pl.pallas_call(
        pack_kernel,
        grid=(grid,),
        in_specs=[pl.BlockSpec((d, bc), lambda i: (0, i))],
        out_specs=pl.BlockSpec(memory_space=pl.ANY),
        out_shape=jax.ShapeDtypeStruct((v, d), jnp.float32),
        scratch_shapes=[
            pltpu.VMEM((bc, d), jnp.float32),
            pltpu.SemaphoreType.DMA,
        ],
    )(table_t)


def _sc_gather_packed(table_p, ids, n, v, d):
    np_ = n // 2
    per_w = np_ // NW
    steps = per_w // CHUNK
    mesh = plsc.VectorSubcoreMesh(core_axis_name="c", subcore_axis_name="s")

    @functools.partial(
        pl.kernel,
        out_type=jax.ShapeDtypeStruct((np_, 2 * d), jnp.float32),
        mesh=mesh,
        scratch_types=[
            pltpu.VMEM((CHUNK,), jnp.int32),
            pltpu.VMEM((CHUNK, d), jnp.float32),
            pltpu.SemaphoreType.DMA,
        ],
        compiler_params=pltpu.CompilerParams(use_tc_tiling_on_sc=False),
    )
    def gather_kernel(tab, idx_hbm, out_hbm, idx_v, rows_v, sem):
        wid = lax.axis_index("s") * NC + lax.axis_index("c")
        base = wid * per_w

        def body(i, carry):
            r0 = base + i * CHUNK
            pltpu.sync_copy(idx_hbm.at[pl.ds(r0, CHUNK)], idx_v)
            pltpu.async_copy(tab.at[idx_v], rows_v, sem).wait()
            pltpu.sync_copy(rows_v, out_hbm.at[pl.ds(r0, CHUNK), pl.ds(0, d)])
            pltpu.sync_copy(idx_hbm.at[pl.ds(np_ + r0, CHUNK)], idx_v)
            pltpu.async_copy(tab.at[idx_v], rows_v, sem).wait()
            pltpu.sync_copy(rows_v, out_hbm.at[pl.ds(r0, CHUNK), pl.ds(d, d)])
            return carry

        lax.fori_loop(0, steps, body, 0)

    return gather_kernel(table_p, ids)


def _tc_project_pair(x2, wa, wb, b, bsz, seq, batch_block):
    np_, k = x2.shape
    h = wa.shape[1]
    rows_per_block = batch_block * seq

    def matmul_kernel(x_ref, wa_ref, wb_ref, b_ref, o_ref):
        x = x_ref[...]
        y0 = jnp.dot(x, wa_ref[...], preferred_element_type=jnp.float32) + b_ref[...]
        y1 = jnp.dot(x, wb_ref[...], preferred_element_type=jnp.float32) + b_ref[...]
        o_ref[0] = y0.reshape(batch_block, seq, h)
        o_ref[1] = y1.reshape(batch_block, seq, h)

    return pl.pallas_call(
        matmul_kernel,
        grid=(np_ // rows_per_block,),
        in_specs=[
            pl.BlockSpec((rows_per_block, k), lambda i: (i, 0)),
            pl.BlockSpec((k, h), lambda i: (0, 0)),
            pl.BlockSpec((k, h), lambda i: (0, 0)),
            pl.BlockSpec((1, h), lambda i: (0, 0)),
        ],
        out_specs=pl.BlockSpec(
            (2, batch_block, seq, h), lambda i: (0, i, 0, 0)
        ),
        out_shape=jax.ShapeDtypeStruct((2, bsz // 2, seq, h), jnp.float32),
    )(x2, wa, wb, b.reshape(1, h))


def kernel(input_ids, token_embed, W, b):
    bsz, seq = input_ids.shape
    n = bsz * seq
    v, d = token_embed.shape
    h = W.shape[1]
    ids = input_ids.reshape(n).astype(jnp.int32)
    table_p = _tc_pack_table(token_embed.T)
    x2 = _sc_gather_packed(table_p, ids, n, v, d)
    zeros = jnp.zeros_like(W)
    wa = jnp.concatenate([W, zeros], axis=0)
    wb = jnp.concatenate([zeros, W], axis=0)
    out4 = _tc_project_pair(x2, wa, wb, b, bsz, seq, batch_block=8)
    return out4.reshape(bsz, seq, h)


# speedup vs baseline: 1.6899x; 1.6899x over previous
import functools

import jax
import jax.numpy as jnp
from jax import lax
from jax.experimental import pallas as pl
from jax.experimental.pallas import tpu as pltpu
from jax.experimental.pallas import tpu_sc as plsc

NC = 2
NS = 16
NW = NC * NS

CHUNK = 400
PACK_BC = 4096


def _tc_pack_table(table_t):
    d, v = table_t.shape
    bc = PACK_BC
    grid = (v + bc - 1) // bc
    tail = v - (grid - 1) * bc
    dn = (((0,), (0,)), ((), ()))
    eye = jnp.eye(d, dtype=jnp.float32)

    def pack_kernel(t_ref, e_ref, o_hbm, y0, y1, s0, s1):
        i = pl.program_id(0)

        def compute():
            yt = lax.dot_general(
                t_ref[...], e_ref[...], dn, preferred_element_type=jnp.float32)
            return jnp.concatenate([yt, jnp.zeros_like(yt)], axis=1)

        def dst(base, rows=bc):
            return o_hbm.at[pl.ds(base, rows), :]

        def ring(y, s, parity):
            @pl.when(jnp.logical_and(i != grid - 1, i % 2 == parity))
            def _():
                @pl.when(i >= 2)
                def _w():
                    pltpu.make_async_copy(y, dst(0), s).wait()
                y[...] = compute()
                pltpu.make_async_copy(y, dst(i * bc), s).start()

        ring(y0, s0, 0)
        ring(y1, s1, 1)

        @pl.when(i == grid - 1)
        def _tail():
            pltpu.make_async_copy(y0, dst(0), s0).wait()
            y0[...] = compute()
            cp = pltpu.make_async_copy(
                y0.at[pl.ds(0, tail)], dst((grid - 1) * bc, tail), s0)
            cp.start()
            cp.wait()
            pltpu.make_async_copy(y1, dst(0), s1).wait()

    return pl.pallas_call(
        pack_kernel,
        grid=(grid,),
        in_specs=[
            pl.BlockSpec((d, bc), lambda i: (0, i)),
            pl.BlockSpec((d, d), lambda i: (0, 0)),
        ],
        out_specs=pl.BlockSpec(memory_space=pl.ANY),
        out_shape=jax.ShapeDtypeStruct((v, 2 * d), jnp.float32),
        scratch_shapes=[
            pltpu.VMEM((bc, 2 * d), jnp.float32),
            pltpu.VMEM((bc, 2 * d), jnp.float32),
            pltpu.SemaphoreType.DMA,
            pltpu.SemaphoreType.DMA,
        ],
        compiler_params=pltpu.CompilerParams(
            fuse_transposed_lhs_in_matmul=True),
    )(table_t, eye)


def _sc_gather(table_z, ids, n):
    dz = table_z.shape[1]
    per_w = n // NW
    steps = per_w // CHUNK
    mesh = plsc.VectorSubcoreMesh(core_axis_name="c", subcore_axis_name="s")

    @functools.partial(
        pl.kernel,
        out_type=jax.ShapeDtypeStruct((n, dz), jnp.float32),
        mesh=mesh,
        scratch_types=[
            pltpu.VMEM((CHUNK,), jnp.int32),
            pltpu.VMEM((CHUNK, dz), jnp.float32),
            pltpu.SemaphoreType.DMA,
        ],
        compiler_params=pltpu.CompilerParams(use_tc_tiling_on_sc=False),
    )
    def gather_kernel(tab, idx_hbm, out_hbm, idx_v, rows_v, sem):
        wid = lax.axis_index("s") * NC + lax.axis_index("c")
        base = wid * per_w

        def body(i, carry):
            r0 = base + i * CHUNK
            pltpu.sync_copy(idx_hbm.at[pl.ds(r0, CHUNK)], idx_v)
            pltpu.async_copy(tab.at[idx_v], rows_v, sem).wait()
            pltpu.sync_copy(rows_v, out_hbm.at[pl.ds(r0, CHUNK)])
            return carry

        lax.fori_loop(0, steps, body, 0)

    return gather_kernel(table_z, ids)


def _tc_project(x2, wz, b, bsz, seq, batch_block):
    n, k = x2.shape
    h = wz.shape[1]
    rows_per_block = batch_block * seq

    def matmul_kernel(x_ref, w_ref, b_ref, o_ref):
        y = jnp.dot(x_ref[...], w_ref[...],
                    preferred_element_type=jnp.float32) + b_ref[...]
        o_ref[...] = y.reshape(batch_block, seq, h)

    return pl.pallas_call(
        matmul_kernel,
        grid=(n // rows_per_block,),
        in_specs=[
            pl.BlockSpec((rows_per_block, k), lambda i: (i, 0)),
            pl.BlockSpec((k, h), lambda i: (0, 0)),
            pl.BlockSpec((1, h), lambda i: (0, 0)),
        ],
        out_specs=pl.BlockSpec(
            (batch_block, seq, h), lambda i: (i, 0, 0)),
        out_shape=jax.ShapeDtypeStruct((bsz, seq, h), jnp.float32),
    )(x2, wz, b.reshape(1, h))


def kernel(input_ids, token_embed, W, b):
    bsz, seq = input_ids.shape
    n = bsz * seq
    v, d = token_embed.shape
    h = W.shape[1]
    ids = input_ids.reshape(n).astype(jnp.int32)
    table_z = _tc_pack_table(token_embed.T)
    x2 = _sc_gather(table_z, ids, n)
    wz = jnp.concatenate([W, jnp.zeros_like(W)], axis=0)
    return _tc_project(x2, wz, b, bsz, seq, batch_block=8)


# speedup vs baseline: 2.1608x; 1.2787x over previous
import functools

import jax
import jax.numpy as jnp
from jax import lax
from jax.experimental import pallas as pl
from jax.experimental.pallas import tpu as pltpu
from jax.experimental.pallas import tpu_sc as plsc

NC = 2
NS = 16
NW = NC * NS

CHUNK = 400
PACK_BC = 8192


def _tc_pack_table(table_t):
    d, v = table_t.shape
    bc = PACK_BC
    grid = (v + bc - 1) // bc
    tail = v - (grid - 1) * bc
    dn = (((0,), (0,)), ((), ()))
    eye = jnp.eye(d, dtype=jnp.float32)

    def pack_kernel(t_ref, e_ref, o_hbm, y0, y1, s0, s1):
        i = pl.program_id(0)

        def compute():
            yt = lax.dot_general(
                t_ref[...], e_ref[...], dn, preferred_element_type=jnp.float32)
            return jnp.concatenate([yt, jnp.zeros_like(yt)], axis=1)

        def dst(base, rows=bc):
            return o_hbm.at[pl.ds(base, rows), :]

        def ring(y, s, parity):
            @pl.when(jnp.logical_and(i != grid - 1, i % 2 == parity))
            def _():
                @pl.when(i >= 2)
                def _w():
                    pltpu.make_async_copy(y, dst(0), s).wait()
                y[...] = compute()
                pltpu.make_async_copy(y, dst(i * bc), s).start()

        ring(y0, s0, 0)
        ring(y1, s1, 1)

        @pl.when(i == grid - 1)
        def _tail():
            pltpu.make_async_copy(y0, dst(0), s0).wait()
            y0[...] = compute()
            cp = pltpu.make_async_copy(
                y0.at[pl.ds(0, tail)], dst((grid - 1) * bc, tail), s0)
            cp.start()
            cp.wait()
            pltpu.make_async_copy(y1, dst(0), s1).wait()

    return pl.pallas_call(
        pack_kernel,
        grid=(grid,),
        in_specs=[
            pl.BlockSpec((d, bc), lambda i: (0, i)),
            pl.BlockSpec((d, d), lambda i: (0, 0)),
        ],
        out_specs=pl.BlockSpec(memory_space=pl.ANY),
        out_shape=jax.ShapeDtypeStruct((v, 2 * d), jnp.float32),
        scratch_shapes=[
            pltpu.VMEM((bc, 2 * d), jnp.float32),
            pltpu.VMEM((bc, 2 * d), jnp.float32),
            pltpu.SemaphoreType.DMA,
            pltpu.SemaphoreType.DMA,
        ],
        compiler_params=pltpu.CompilerParams(
            fuse_transposed_lhs_in_matmul=True),
    )(table_t, eye)


def _sc_gather(table_z, ids, n):
    dz = table_z.shape[1]
    per_w = n // NW
    steps = per_w // CHUNK
    mesh = plsc.VectorSubcoreMesh(core_axis_name="c", subcore_axis_name="s")

    @functools.partial(
        pl.kernel,
        out_type=jax.ShapeDtypeStruct((n, dz), jnp.float32),
        mesh=mesh,
        scratch_types=[
            pltpu.VMEM((CHUNK,), jnp.int32),
            pltpu.VMEM((CHUNK,), jnp.int32),
            pltpu.VMEM((CHUNK, dz), jnp.float32),
            pltpu.VMEM((CHUNK, dz), jnp.float32),
            pltpu.SemaphoreType.DMA,
            pltpu.SemaphoreType.DMA,
            pltpu.SemaphoreType.DMA,
            pltpu.SemaphoreType.DMA,
        ],
        compiler_params=pltpu.CompilerParams(use_tc_tiling_on_sc=False),
    )
    def gather_kernel(tab, idx_hbm, out_hbm,
                      idx0, idx1, rows0, rows1, sg0, sg1, sw0, sw1):
        wid = lax.axis_index("s") * NC + lax.axis_index("c")
        base = wid * per_w
        idx = (idx0, idx1)
        rows = (rows0, rows1)
        sg = (sg0, sg1)
        sw = (sw0, sw1)

        def fire(c):
            p = c % 2
            pltpu.sync_copy(idx_hbm.at[pl.ds(base + c * CHUNK, CHUNK)], idx[p])
            return pltpu.async_copy(tab.at[idx[p]], rows[p], sg[p])

        g = [None] * steps
        w = [None] * steps
        g[0] = fire(0)
        for i in range(steps):
            p = i % 2
            if i + 1 < steps:
                if i >= 1:
                    w[i - 1].wait()
                g[i + 1] = fire(i + 1)
            g[i].wait()
            w[i] = pltpu.make_async_copy(
                rows[p], out_hbm.at[pl.ds(base + i * CHUNK, CHUNK)], sw[p])
            w[i].start()
        w[steps - 2].wait()
        w[steps - 1].wait()

    return gather_kernel(table_z, ids)


def _tc_project(x2, wz, b, bsz, seq, batch_block):
    n, k = x2.shape
    h = wz.shape[1]
    rows_per_block = batch_block * seq

    def matmul_kernel(x_ref, w_ref, b_ref, o_ref):
        y = jnp.dot(x_ref[...], w_ref[...],
                    preferred_element_type=jnp.float32) + b_ref[...]
        o_ref[...] = y.reshape(batch_block, seq, h)

    return pl.pallas_call(
        matmul_kernel,
        grid=(n // rows_per_block,),
        in_specs=[
            pl.BlockSpec((rows_per_block, k), lambda i: (i, 0)),
            pl.BlockSpec((k, h), lambda i: (0, 0)),
            pl.BlockSpec((1, h), lambda i: (0, 0)),
        ],
        out_specs=pl.BlockSpec(
            (batch_block, seq, h), lambda i: (i, 0, 0)),
        out_shape=jax.ShapeDtypeStruct((bsz, seq, h), jnp.float32),
    )(x2, wz, b.reshape(1, h))


def kernel(input_ids, token_embed, W, b):
    bsz, seq = input_ids.shape
    n = bsz * seq
    v, d = token_embed.shape
    h = W.shape[1]
    ids = input_ids.reshape(n).astype(jnp.int32)
    table_z = _tc_pack_table(token_embed.T)
    x2 = _sc_gather(table_z, ids, n)
    wz = jnp.concatenate([W, jnp.zeros_like(W)], axis=0)
    return _tc_project(x2, wz, b, bsz, seq, batch_block=16)


# speedup vs baseline: 2.2881x; 1.0589x over previous
import functools

import jax
import jax.numpy as jnp
from jax import lax
from jax.experimental import pallas as pl
from jax.experimental.pallas import tpu as pltpu
from jax.experimental.pallas import tpu_sc as plsc

NC = 2
NS = 16
NW = NC * NS

CHUNK = 400
PACK_BC = 16384


def _tc_pack_table(table_t):
    d, v = table_t.shape
    bc = PACK_BC
    grid = (v + bc - 1) // bc
    tail = v - (grid - 1) * bc
    dn = (((0,), (0,)), ((), ()))
    eye = jnp.eye(d, dtype=jnp.float32)

    def pack_kernel(t_ref, e_ref, o_hbm, y0, y1, s0, s1):
        i = pl.program_id(0)

        def compute():
            yt = lax.dot_general(
                t_ref[...], e_ref[...], dn, preferred_element_type=jnp.float32)
            return jnp.concatenate([yt, jnp.zeros_like(yt)], axis=1)

        def dst(base, rows=bc):
            return o_hbm.at[pl.ds(base, rows), :]

        def ring(y, s, parity):
            @pl.when(jnp.logical_and(i != grid - 1, i % 2 == parity))
            def _():
                @pl.when(i >= 2)
                def _w():
                    pltpu.make_async_copy(y, dst(0), s).wait()
                y[...] = compute()
                pltpu.make_async_copy(y, dst(i * bc), s).start()

        ring(y0, s0, 0)
        ring(y1, s1, 1)

        @pl.when(i == grid - 1)
        def _tail():
            pltpu.make_async_copy(y0, dst(0), s0).wait()
            y0[...] = compute()
            cp = pltpu.make_async_copy(
                y0.at[pl.ds(0, tail)], dst((grid - 1) * bc, tail), s0)
            cp.start()
            cp.wait()
            pltpu.make_async_copy(y1, dst(0), s1).wait()

    return pl.pallas_call(
        pack_kernel,
        grid=(grid,),
        in_specs=[
            pl.BlockSpec((d, bc), lambda i: (0, i)),
            pl.BlockSpec((d, d), lambda i: (0, 0)),
        ],
        out_specs=pl.BlockSpec(memory_space=pl.ANY),
        out_shape=jax.ShapeDtypeStruct((v, 2 * d), jnp.float32),
        scratch_shapes=[
            pltpu.VMEM((bc, 2 * d), jnp.float32),
            pltpu.VMEM((bc, 2 * d), jnp.float32),
            pltpu.SemaphoreType.DMA,
            pltpu.SemaphoreType.DMA,
        ],
        compiler_params=pltpu.CompilerParams(
            fuse_transposed_lhs_in_matmul=True),
    )(table_t, eye)


def _sc_gather(table_z, ids, n):
    dz = table_z.shape[1]
    per_w = n // NW
    steps = per_w // CHUNK
    mesh = plsc.VectorSubcoreMesh(core_axis_name="c", subcore_axis_name="s")

    @functools.partial(
        pl.kernel,
        out_type=jax.ShapeDtypeStruct((n, dz), jnp.float32),
        mesh=mesh,
        scratch_types=[
            pltpu.VMEM((CHUNK,), jnp.int32),
            pltpu.VMEM((CHUNK,), jnp.int32),
            pltpu.VMEM((CHUNK, dz), jnp.float32),
            pltpu.VMEM((CHUNK, dz), jnp.float32),
            pltpu.SemaphoreType.DMA,
            pltpu.SemaphoreType.DMA,
            pltpu.SemaphoreType.DMA,
            pltpu.SemaphoreType.DMA,
        ],
        compiler_params=pltpu.CompilerParams(use_tc_tiling_on_sc=False),
    )
    def gather_kernel(tab, idx_hbm, out_hbm,
                      idx0, idx1, rows0, rows1, sg0, sg1, sw0, sw1):
        wid = lax.axis_index("s") * NC + lax.axis_index("c")
        base = wid * per_w
        idx = (idx0, idx1)
        rows = (rows0, rows1)
        sg = (sg0, sg1)
        sw = (sw0, sw1)

        def fire(c):
            p = c % 2
            pltpu.sync_copy(idx_hbm.at[pl.ds(base + c * CHUNK, CHUNK)], idx[p])
            return pltpu.async_copy(tab.at[idx[p]], rows[p], sg[p])

        g = [None] * steps
        w = [None] * steps
        g[0] = fire(0)
        for i in range(steps):
            p = i % 2
            if i + 1 < steps:
                if i >= 1:
                    w[i - 1].wait()
                g[i + 1] = fire(i + 1)
            g[i].wait()
            w[i] = pltpu.make_async_copy(
                rows[p], out_hbm.at[pl.ds(base + i * CHUNK, CHUNK)], sw[p])
            w[i].start()
        w[steps - 2].wait()
        w[steps - 1].wait()

    return gather_kernel(table_z, ids)


def _tc_project(x2, wz, b, bsz, seq, batch_block):
    n, k = x2.shape
    h = wz.shape[1]
    rows_per_block = batch_block * seq

    def matmul_kernel(x_ref, w_ref, b_ref, o_ref):
        y = jnp.dot(x_ref[...], w_ref[...],
                    preferred_element_type=jnp.float32) + b_ref[...]
        o_ref[...] = y.reshape(batch_block, seq, h)

    return pl.pallas_call(
        matmul_kernel,
        grid=(n // rows_per_block,),
        in_specs=[
            pl.BlockSpec((rows_per_block, k), lambda i: (i, 0)),
            pl.BlockSpec((k, h), lambda i: (0, 0)),
            pl.BlockSpec((1, h), lambda i: (0, 0)),
        ],
        out_specs=pl.BlockSpec(
            (batch_block, seq, h), lambda i: (i, 0, 0)),
        out_shape=jax.ShapeDtypeStruct((bsz, seq, h), jnp.float32),
    )(x2, wz, b.reshape(1, h))


def kernel(input_ids, token_embed, W, b):
    bsz, seq = input_ids.shape
    n = bsz * seq
    v, d = token_embed.shape
    h = W.shape[1]
    ids = input_ids.reshape(n).astype(jnp.int32)
    table_z = _tc_pack_table(token_embed.T)
    x2 = _sc_gather(table_z, ids, n)
    wz = jnp.concatenate([W, jnp.zeros_like(W)], axis=0)
    return _tc_project(x2, wz, b, bsz, seq, batch_block=32)


# speedup vs baseline: 2.3047x; 1.0073x over previous
import functools

import jax
import jax.numpy as jnp
from jax import lax
from jax.experimental import pallas as pl
from jax.experimental.pallas import tpu as pltpu
from jax.experimental.pallas import tpu_sc as plsc

NC = 2
NS = 16
NW = NC * NS

CHUNK = 400
PACK_BC = 16384


def _tc_pack_table(table_t):
    d, v = table_t.shape
    bc = PACK_BC
    grid = (v + bc - 1) // bc
    tail = v - (grid - 1) * bc
    dn = (((0,), (0,)), ((), ()))
    eye = jnp.eye(d, dtype=jnp.bfloat16)

    def pack_kernel(t_ref, e_ref, o_hbm, y0, y1, s0, s1):
        i = pl.program_id(0)

        def compute():
            yt = lax.dot_general(
                t_ref[...].astype(jnp.bfloat16), e_ref[...], dn,
                preferred_element_type=jnp.float32)
            return jnp.concatenate([yt, jnp.zeros_like(yt)], axis=1)

        def dst(base, rows=bc):
            return o_hbm.at[pl.ds(base, rows), :]

        def ring(y, s, parity):
            @pl.when(jnp.logical_and(i != grid - 1, i % 2 == parity))
            def _():
                @pl.when(i >= 2)
                def _w():
                    pltpu.make_async_copy(y, dst(0), s).wait()
                y[...] = compute()
                pltpu.make_async_copy(y, dst(i * bc), s).start()

        ring(y0, s0, 0)
        ring(y1, s1, 1)

        @pl.when(i == grid - 1)
        def _tail():
            pltpu.make_async_copy(y0, dst(0), s0).wait()
            y0[...] = compute()
            cp = pltpu.make_async_copy(
                y0.at[pl.ds(0, tail)], dst((grid - 1) * bc, tail), s0)
            cp.start()
            cp.wait()
            pltpu.make_async_copy(y1, dst(0), s1).wait()

    return pl.pallas_call(
        pack_kernel,
        grid=(grid,),
        in_specs=[
            pl.BlockSpec((d, bc), lambda i: (0, i)),
            pl.BlockSpec((d, d), lambda i: (0, 0)),
        ],
        out_specs=pl.BlockSpec(memory_space=pl.ANY),
        out_shape=jax.ShapeDtypeStruct((v, 2 * d), jnp.float32),
        scratch_shapes=[
            pltpu.VMEM((bc, 2 * d), jnp.float32),
            pltpu.VMEM((bc, 2 * d), jnp.float32),
            pltpu.SemaphoreType.DMA,
            pltpu.SemaphoreType.DMA,
        ],
        compiler_params=pltpu.CompilerParams(
            fuse_transposed_lhs_in_matmul=True),
    )(table_t, eye)


def _sc_gather(table_z, ids, n):
    dz = table_z.shape[1]
    per_w = n // NW
    steps = per_w // CHUNK
    mesh = plsc.VectorSubcoreMesh(core_axis_name="c", subcore_axis_name="s")

    @functools.partial(
        pl.kernel,
        out_type=jax.ShapeDtypeStruct((n, dz), jnp.float32),
        mesh=mesh,
        scratch_types=[
            pltpu.VMEM((CHUNK,), jnp.int32),
            pltpu.VMEM((CHUNK,), jnp.int32),
            pltpu.VMEM((CHUNK, dz), jnp.float32),
            pltpu.VMEM((CHUNK, dz), jnp.float32),
            pltpu.SemaphoreType.DMA,
            pltpu.SemaphoreType.DMA,
            pltpu.SemaphoreType.DMA,
            pltpu.SemaphoreType.DMA,
        ],
        compiler_params=pltpu.CompilerParams(use_tc_tiling_on_sc=False),
    )
    def gather_kernel(tab, idx_hbm, out_hbm,
                      idx0, idx1, rows0, rows1, sg0, sg1, sw0, sw1):
        wid = lax.axis_index("s") * NC + lax.axis_index("c")
        base = wid * per_w
        idx = (idx0, idx1)
        rows = (rows0, rows1)
        sg = (sg0, sg1)
        sw = (sw0, sw1)

        def fire(c):
            p = c % 2
            pltpu.sync_copy(idx_hbm.at[pl.ds(base + c * CHUNK, CHUNK)], idx[p])
            return pltpu.async_copy(tab.at[idx[p]], rows[p], sg[p])

        g = [None] * steps
        w = [None] * steps
        g[0] = fire(0)
        for i in range(steps):
            p = i % 2
            if i + 1 < steps:
                if i >= 1:
                    w[i - 1].wait()
                g[i + 1] = fire(i + 1)
            g[i].wait()
            w[i] = pltpu.make_async_copy(
                rows[p], out_hbm.at[pl.ds(base + i * CHUNK, CHUNK)], sw[p])
            w[i].start()
        w[steps - 2].wait()
        w[steps - 1].wait()

    return gather_kernel(table_z, ids)


def _tc_project(x2, wz, b, bsz, seq, batch_block):
    n, k = x2.shape
    h = wz.shape[1]
    rows_per_block = batch_block * seq

    def matmul_kernel(x_ref, w_ref, b_ref, o_ref):
        y = jnp.dot(x_ref[...], w_ref[...],
                    preferred_element_type=jnp.float32) + b_ref[...]
        o_ref[...] = y.reshape(batch_block, seq, h)

    return pl.pallas_call(
        matmul_kernel,
        grid=(n // rows_per_block,),
        in_specs=[
            pl.BlockSpec((rows_per_block, k), lambda i: (i, 0)),
            pl.BlockSpec((k, h), lambda i: (0, 0)),
            pl.BlockSpec((1, h), lambda i: (0, 0)),
        ],
        out_specs=pl.BlockSpec(
            (batch_block, seq, h), lambda i: (i, 0, 0)),
        out_shape=jax.ShapeDtypeStruct((bsz, seq, h), jnp.float32),
    )(x2, wz, b.reshape(1, h))


def kernel(input_ids, token_embed, W, b):
    bsz, seq = input_ids.shape
    n = bsz * seq
    v, d = token_embed.shape
    h = W.shape[1]
    ids = input_ids.reshape(n).astype(jnp.int32)
    table_z = _tc_pack_table(token_embed.T)
    x2 = _sc_gather(table_z, ids, n)
    wz = jnp.concatenate([W, jnp.zeros_like(W)], axis=0)
    return _tc_project(x2, wz, b, bsz, seq, batch_block=32)
